# Initial kernel scaffold; baseline (speedup 1.0000x reference)
import jax
import jax.numpy as jnp
from jax.experimental import pallas as pl


def kernel(input_coords, feat_params):
    raise NotImplementedError("write your pallas kernel here")


# speedup vs baseline: 2.2035x; 2.2035x over previous
import functools

import numpy as np
import jax
import jax.numpy as jnp
from jax import lax
from jax.experimental import pallas as pl
from jax.experimental.pallas import tpu as pltpu
from jax.experimental.pallas import tpu_sc as plsc

_HASH_CAP = 10000
_RES = 8
_VS = np.float32(0.01)
_INV_VS = np.float32(1.0) / _VS
_P = 128
_G = _P // 16
_NW = 32
_HX, _HY, _HZ = 73856093, 19349663, 83492791


def _mod_cap(h, cap):
    c1 = np.int32((1 << 16) % cap)
    h = (h >> 16) * c1 + (h & 0xFFFF)
    h = (h >> 16) * c1 + (h & 0xFFFF)
    q = (h.astype(jnp.float32) * np.float32(1.0 / cap)).astype(jnp.int32)
    r = h - q * np.int32(cap)
    r = jnp.where(r < 0, r + np.int32(cap), r)
    return jnp.where(r >= cap, r - np.int32(cap), r)


_NFP = 8


def _tec_body(stride, nf, xs_hbm, ys_hbm, zs_hbm, table_hbm, feats_hbm,
              df_hbm, xv, yv, zv, idxb, fbuf, wbuf, ofeats, odf, gsem):
    n = feats_hbm.shape[0]
    num_chunks = (n + _P - 1) // _P
    wid = lax.axis_index("c") * 16 + lax.axis_index("s")
    lanes = lax.iota(jnp.int32, 16)
    nmine = (num_chunks - wid + _NW - 1) // _NW

    def chunk_body(g, carry):
        chunk = wid + g * _NW
        base = jnp.minimum(chunk * _P, n - _P)
        pltpu.sync_copy(xs_hbm.at[pl.ds(base, _P)], xv)
        pltpu.sync_copy(ys_hbm.at[pl.ds(base, _P)], yv)
        pltpu.sync_copy(zs_hbm.at[pl.ds(base, _P)], zv)

        for p in range(_G):
            s = pl.ds(p * 16, 16)
            coords = [xv[s], yv[s], zv[s]]
            blk = []
            loc = []
            wgt = []
            for d, (hmul, lmul) in enumerate(((_HX, 64), (_HY, 8), (_HZ, 1))):
                gcoord = coords[d] / _VS
                b = gcoord.astype(jnp.int32)
                frac = gcoord - b.astype(jnp.float32)
                b1 = b + 1
                blk.append(((b >> 3) * hmul, (b1 >> 3) * hmul))
                loc.append(((b & 7) * lmul, (b1 & 7) * lmul))
                wgt.append((np.float32(1.0) - frac, frac))
            for t in range(2):
                wbuf[0 + t, s] = wgt[0][t]
                wbuf[2 + t, s] = wgt[1][t]
                wbuf[4 + t, s] = wgt[2][t]
            for i in range(2):
                for j in range(2):
                    hxy = blk[0][i] ^ blk[1][j]
                    lxy = loc[0][i] + loc[1][j]
                    for k in range(2):
                        c = i * 4 + j * 2 + k
                        h = _mod_cap(hxy ^ blk[2][k], _HASH_CAP)
                        idxb[c, s] = h * stride + (lxy + loc[2][k])

        cps = [pltpu.async_copy(table_hbm.at[idxb.at[c]], fbuf.at[c], gsem)
               for c in range(8)]
        for cp in cps:
            cp.wait()

        for p in range(_G):
            s = pl.ds(p * 16, 16)
            rows = lanes + np.int32(p * 16)
            wx0 = wbuf[0, s]; wx1 = wbuf[1, s]
            wy0 = wbuf[2, s]; wy1 = wbuf[3, s]
            wz0 = wbuf[4, s]; wz1 = wbuf[5, s]
            wx = (wx0, wx1); wy = (wy0, wy1); wz = (wz0, wz1)
            wxy = [wx[i] * wy[j] for i in range(2) for j in range(2)]
            wyz = [wy[j] * wz[k] for j in range(2) for k in range(2)]
            wxz = [wx[i] * wz[k] for i in range(2) for k in range(2)]
            w8 = [wxy[i * 2 + j] * wz[k]
                  for i in range(2) for j in range(2) for k in range(2)]
            for jf in range(nf):
                jv = jnp.full((16,), jf, jnp.int32)
                f = [plsc.load_gather(
                        fbuf, [jnp.full((16,), c, jnp.int32), rows, jv])
                     for c in range(8)]
                acc = w8[0] * f[0]
                for c in range(1, 8):
                    acc = acc + w8[c] * f[c]
                dfx = wyz[0] * (f[4] - f[0])
                for t in range(1, 4):
                    dfx = dfx + wyz[t] * (f[4 + t] - f[t])
                dfy = wxz[0] * (f[2] - f[0])
                for t in range(1, 4):
                    i, k = divmod(t, 2)
                    dfy = dfy + wxz[t] * (f[i * 4 + 2 + k] - f[i * 4 + k])
                dfz = wxy[0] * (f[1] - f[0])
                for t in range(1, 4):
                    i, j = divmod(t, 2)
                    dfz = dfz + wxy[t] * (f[i * 4 + j * 2 + 1] - f[i * 4 + j * 2])
                plsc.store_scatter(ofeats, [rows, jv], acc)
                for dd, val in ((0, dfx), (1, dfy), (2, dfz)):
                    plsc.store_scatter(
                        odf, [rows, jnp.full((16,), 3 * jf + dd, jnp.int32)],
                        val * _INV_VS)

        pltpu.sync_copy(ofeats, feats_hbm.at[pl.ds(base, _P)])
        pltpu.sync_copy(odf, df_hbm.at[pl.ds(base, _P)])
        return carry

    lax.fori_loop(0, nmine, chunk_body, 0)


@functools.lru_cache(maxsize=None)
def _build(n, stride, nf):
    mesh = plsc.VectorSubcoreMesh(core_axis_name="c", subcore_axis_name="s")
    return pl.kernel(
        functools.partial(_tec_body, stride, nf),
        out_type=(
            jax.ShapeDtypeStruct((n, nf), jnp.float32),
            jax.ShapeDtypeStruct((n, 3 * nf), jnp.float32),
        ),
        mesh=mesh,
        scratch_types=[
            pltpu.VMEM((_P,), jnp.float32),
            pltpu.VMEM((_P,), jnp.float32),
            pltpu.VMEM((_P,), jnp.float32),
            pltpu.VMEM((8, _P), jnp.int32),
            pltpu.VMEM((8, _P, _NFP), jnp.float32),
            pltpu.VMEM((6, _P), jnp.float32),
            pltpu.VMEM((_P, nf), jnp.float32),
            pltpu.VMEM((_P, 3 * nf), jnp.float32),
            pltpu.SemaphoreType.DMA,
        ],
        compiler_params=pltpu.CompilerParams(
            needs_layout_passes=False, use_tc_tiling_on_sc=False),
    )


def kernel(input_coords, feat_params):
    n = input_coords.shape[0]
    cap, stride, nf = feat_params.shape
    table = jnp.pad(feat_params.reshape(cap * stride, nf),
                    ((0, 0), (0, _NFP - nf)))
    xs = input_coords[:, 0]
    ys = input_coords[:, 1]
    zs = input_coords[:, 2]
    feats, df = _build(n, stride, nf)(xs, ys, zs, table)
    return (feats,
            df.reshape(n, nf, 3),
            jnp.ones((n,), dtype=jnp.bool_))
